# baseline (speedup 1.0000x reference)
import functools

import jax
import jax.numpy as jnp
from jax import lax
from jax.experimental import pallas as pl
from jax.experimental.pallas import tpu as pltpu
from jax.experimental.pallas import tpu_sc as plsc

D_NUM = 64
D_EMB = 128
D_OUT = D_NUM + 3 * D_EMB
BT = 128


@functools.lru_cache(maxsize=None)
def _make_kernel(B: int, L: int):
    info = plsc.get_sparse_core_info()
    NC, NS = info.num_cores, info.num_subcores
    NW = NC * NS
    assert B % (NW * BT) == 0
    bt_per_w = B // (NW * BT)
    n_lt = (L + 7) // 8

    mesh = plsc.VectorSubcoreMesh(core_axis_name="c", subcore_axis_name="s")

    @functools.partial(
        pl.kernel,
        mesh=mesh,
        out_type=jax.ShapeDtypeStruct((L, D_OUT, B), jnp.float32),
        scratch_types=[
            pltpu.VMEM((8, BT), jnp.int32),
            pltpu.VMEM((8, BT), jnp.int32),
            pltpu.VMEM((8, BT), jnp.int32),
            pltpu.VMEM((BT, D_EMB), jnp.float32),
            pltpu.VMEM((BT, D_EMB), jnp.float32),
            pltpu.VMEM((BT, D_EMB), jnp.float32),
            pltpu.VMEM((3 * D_EMB, BT), jnp.float32),
            pltpu.VMEM((D_NUM, BT), jnp.float32),
            pltpu.SemaphoreType.DMA,
        ],
        compiler_params=pltpu.CompilerParams(needs_layout_passes=False),
    )
    def k(numeric, wp_id, gl_id, ac_id, w_wp, w_gl, w_ac, out,
          wi, gi, ai, wpv, glv, acv, ebuf, nbuf, sem):
        wid = lax.axis_index("s") * NC + lax.axis_index("c")
        nb0 = wid * bt_per_w * BT

        iota = lax.iota(jnp.int32, 16)
        lanes = [iota + 16 * kk for kk in range(8)]

        def do_block(b0, l0, lsz, lslc):
            pltpu.sync_copy(wp_id.at[lslc, pl.ds(b0, BT)], wi.at[pl.ds(0, lsz), :])
            pltpu.sync_copy(gl_id.at[lslc, pl.ds(b0, BT)], gi.at[pl.ds(0, lsz), :])
            pltpu.sync_copy(ac_id.at[lslc, pl.ds(b0, BT)], ai.at[pl.ds(0, lsz), :])

            def per_l(lr, carry2):
                cw = pltpu.async_copy(w_wp.at[wi.at[lr, :]], wpv, sem)
                cg = pltpu.async_copy(w_gl.at[gi.at[lr, :]], glv, sem)
                ca = pltpu.async_copy(w_ac.at[ai.at[lr, :]], acv, sem)
                cn = pltpu.async_copy(
                    numeric.at[l0 + lr, :, pl.ds(b0, BT)], nbuf, sem)
                cw.wait()
                cg.wait()
                ca.wait()
                cn.wait()
                pltpu.sync_copy(
                    nbuf, out.at[l0 + lr, pl.ds(0, D_NUM), pl.ds(b0, BT)])

                def xpose(rg, carry3):
                    r0 = rg * 8
                    for rr in range(8):
                        r = r0 + rr
                        col = jnp.broadcast_to(r, (16,))
                        for t, src in enumerate((wpv, glv, acv)):
                            for kk in range(8):
                                v = plsc.load_gather(src, [lanes[kk], col])
                                ebuf[t * D_EMB + r, pl.ds(16 * kk, 16)] = v
                    return carry3

                lax.fori_loop(0, D_EMB // 8, xpose, 0)
                pltpu.sync_copy(
                    ebuf,
                    out.at[l0 + lr, pl.ds(D_NUM, 3 * D_EMB), pl.ds(b0, BT)])
                return carry2

            lax.fori_loop(0, lsz, per_l, 0)

        def body(u, carry):
            bb = u // (n_lt - 1)
            lt = u % (n_lt - 1)
            l0 = pl.multiple_of(lt * 8, 8)
            b0 = pl.multiple_of(nb0 + bb * BT, BT)
            do_block(b0, l0, 8, pl.ds(l0, 8))
            return carry

        lax.fori_loop(0, bt_per_w * (n_lt - 1), body, 0)

        def tail(u, carry):
            b0 = pl.multiple_of(nb0 + u * BT, BT)
            do_block(b0, (n_lt - 1) * 8, L - (n_lt - 1) * 8,
                     pl.ds((n_lt - 1) * 8, L - (n_lt - 1) * 8))
            return carry

        lax.fori_loop(0, bt_per_w, tail, 0)

    return k


def kernel(numeric, waypoint_id, final_goal_id, action_id, W_wp, W_gl, W_ac):
    B, L, _ = numeric.shape
    num_t = jnp.transpose(numeric, (1, 2, 0))
    wi = jnp.transpose(waypoint_id.astype(jnp.int32), (1, 0))
    gi = jnp.transpose(final_goal_id.astype(jnp.int32), (1, 0))
    ai = jnp.transpose(action_id.astype(jnp.int32), (1, 0))
    out_t = _make_kernel(B, L)(num_t, wi, gi, ai, W_wp, W_gl, W_ac)
    return jnp.transpose(out_t, (2, 0, 1))


# speedup vs baseline: 2.0562x; 2.0562x over previous
import functools

import jax
import jax.numpy as jnp
from jax import lax
from jax.experimental import pallas as pl
from jax.experimental.pallas import tpu as pltpu
from jax.experimental.pallas import tpu_sc as plsc

D_NUM = 64
D_EMB = 128
D_OUT = D_NUM + 3 * D_EMB
NB = 2


@functools.lru_cache(maxsize=None)
def _make_kernel(B: int, L: int):
    info = plsc.get_sparse_core_info()
    NC, NS = info.num_cores, info.num_subcores
    NW = NC * NS
    assert B % (NW * NB) == 0
    per_w = B // NW
    n_iter = per_w // NB

    mesh = plsc.VectorSubcoreMesh(core_axis_name="c", subcore_axis_name="s")

    @functools.partial(
        pl.kernel,
        mesh=mesh,
        out_type=jax.ShapeDtypeStruct((B, L, D_OUT), jnp.float32),
        scratch_types=[
            pltpu.VMEM((NB, 128), jnp.int32),
            pltpu.VMEM((NB, 128), jnp.int32),
            pltpu.VMEM((NB, 128), jnp.int32),
            pltpu.VMEM((NB, L, D_EMB), jnp.float32),
            pltpu.VMEM((NB, L, D_EMB), jnp.float32),
            pltpu.VMEM((NB, L, D_EMB), jnp.float32),
            pltpu.VMEM((NB, L, D_EMB), jnp.float32),
            pltpu.VMEM((NB, L, D_EMB), jnp.float32),
            pltpu.VMEM((NB, L, D_EMB), jnp.float32),
            pltpu.VMEM((NB, L, D_NUM), jnp.float32),
            pltpu.SemaphoreType.DMA,
        ],
    )
    def k(numeric, wp_id, gl_id, ac_id, w_wp, w_gl, w_ac, out,
          wi, gi, ai, wpv, glv, acv, p0, p1, p2, p3, sem):
        wid = lax.axis_index("s") * NC + lax.axis_index("c")
        w_base = wid * per_w
        lidx = pl.ds(0, L)

        def body(i, carry):
            b0 = w_base + i * NB
            pltpu.sync_copy(wp_id.at[pl.ds(b0, NB), :], wi)
            pltpu.sync_copy(gl_id.at[pl.ds(b0, NB), :], gi)
            pltpu.sync_copy(ac_id.at[pl.ds(b0, NB), :], ai)
            cps = [pltpu.async_copy(numeric.at[pl.ds(b0, NB), :, :], p0, sem)]
            for j in range(NB):
                cps += [
                    pltpu.async_copy(w_wp.at[wi.at[j, lidx]], wpv.at[j], sem),
                    pltpu.async_copy(w_gl.at[gi.at[j, lidx]], glv.at[j], sem),
                    pltpu.async_copy(w_ac.at[ai.at[j, lidx]], acv.at[j], sem),
                ]
            for c in cps:
                c.wait()

            def shuffle(l, carry2):
                for j in range(NB):
                    for v in range(4):
                        s = pl.ds(16 * v, 16)
                        d = pl.ds(D_NUM + 16 * v, 16)
                        p0[j, l, d] = wpv[j, l, s]
                        p1[j, l, s] = wpv[j, l, d]
                        p1[j, l, d] = glv[j, l, s]
                        p2[j, l, s] = glv[j, l, d]
                        p2[j, l, d] = acv[j, l, s]
                        p3[j, l, s] = acv[j, l, d]
                return carry2

            lax.fori_loop(0, L, shuffle, 0)
            rows = pl.ds(b0, NB)
            pltpu.sync_copy(p0, out.at[rows, :, pl.ds(0, 128)])
            pltpu.sync_copy(p1, out.at[rows, :, pl.ds(128, 128)])
            pltpu.sync_copy(p2, out.at[rows, :, pl.ds(256, 128)])
            pltpu.sync_copy(p3, out.at[rows, :, pl.ds(384, 64)])
            return carry

        lax.fori_loop(0, n_iter, body, 0)

    return k


def kernel(numeric, waypoint_id, final_goal_id, action_id, W_wp, W_gl, W_ac):
    B, L, d_num = numeric.shape
    wi = jnp.pad(waypoint_id.astype(jnp.int32), ((0, 0), (0, 128 - L)))
    gi = jnp.pad(final_goal_id.astype(jnp.int32), ((0, 0), (0, 128 - L)))
    ai = jnp.pad(action_id.astype(jnp.int32), ((0, 0), (0, 128 - L)))
    num_p = jnp.pad(numeric, ((0, 0), (0, 0), (0, D_EMB - d_num)))
    return _make_kernel(B, L)(num_p, wi, gi, ai, W_wp, W_gl, W_ac)
